# Initial kernel scaffold; baseline (speedup 1.0000x reference)
import jax
import jax.numpy as jnp
from jax.experimental import pallas as pl


def kernel(x, enc_w, enc_b, dec_w, dec_b):
    raise NotImplementedError("write your pallas kernel here")


# speedup vs baseline: 18.5423x; 18.5423x over previous
import functools

import jax
import jax.numpy as jnp
from jax.experimental import pallas as pl
from jax.experimental.pallas import tpu as pltpu

C = 1024
K = 4096
TOPK = 20
T = 256


def _body(x_ref, ew_ref, eb_ref, dw_ref, db_ref, z_ref, xh_ref):
    xb = x_ref[...]
    ew = ew_ref[...]
    zpre = jax.lax.dot_general(
        xb, ew, (((1,), (1,)), ((), ())),
        preferred_element_type=jnp.float32)
    zpre = zpre + eb_ref[...]

    work = zpre
    for _ in range(TOPK - 1):
        m = jnp.max(work, axis=1, keepdims=True)
        work = jnp.where(work == m, -jnp.inf, work)
    thresh = jnp.max(work, axis=1, keepdims=True)

    z = jnp.where(zpre >= thresh, jnp.maximum(zpre, 0.0), 0.0)
    z_ref[...] = z

    dw = dw_ref[...]
    xh = jax.lax.dot_general(
        z.astype(jnp.bfloat16), dw, (((1,), (1,)), ((), ())),
        preferred_element_type=jnp.float32)
    xh_ref[...] = xh + db_ref[...]


@functools.partial(jax.jit, static_argnums=())
def kernel(x, enc_w, enc_b, dec_w, dec_b):
    B, N, _ = x.shape
    M = B * N
    xf = x.reshape(M, C).astype(jnp.bfloat16)
    ew = enc_w.astype(jnp.bfloat16)
    dw = dec_w.astype(jnp.bfloat16)
    eb = enc_b.reshape(1, K)
    db = dec_b.reshape(1, C)

    z, xh = pl.pallas_call(
        _body,
        grid=(M // T,),
        in_specs=[
            pl.BlockSpec((T, C), lambda i: (i, 0)),
            pl.BlockSpec((K, C), lambda i: (0, 0)),
            pl.BlockSpec((1, K), lambda i: (0, 0)),
            pl.BlockSpec((C, K), lambda i: (0, 0)),
            pl.BlockSpec((1, C), lambda i: (0, 0)),
        ],
        out_specs=[
            pl.BlockSpec((T, K), lambda i: (i, 0)),
            pl.BlockSpec((T, C), lambda i: (i, 0)),
        ],
        out_shape=[
            jax.ShapeDtypeStruct((M, K), jnp.float32),
            jax.ShapeDtypeStruct((M, C), jnp.float32),
        ],
        compiler_params=pltpu.CompilerParams(
            dimension_semantics=("parallel",)),
    )(xf, ew, eb, dw, db)
    return z.reshape(B, N, K), xh.reshape(B, N, C)


# speedup vs baseline: 26.3431x; 1.4207x over previous
import jax
import jax.numpy as jnp
from jax.experimental import pallas as pl
from jax.experimental.pallas import tpu as pltpu

C = 1024
K = 4096
TOPK = 20
T = 256
NSLICE = 16
W = K // NSLICE
NEG = float("-inf")


def _row_kth_full(zpre):
    w = zpre
    for _ in range(TOPK - 1):
        m = jnp.max(w, axis=1, keepdims=True)
        w = jnp.where(w == m, NEG, w)
    return jnp.max(w, axis=1, keepdims=True)


def _row_kth_hier(zpre):
    slices = [zpre[:, i * W:(i + 1) * W] for i in range(NSLICE)]

    def tree_max(ss):
        m = ss[0]
        for s in ss[1:]:
            m = jnp.maximum(m, s)
        return m

    cm = [tree_max(slices)]
    for _ in range(3):
        slices = [jnp.where(s == cm[-1], NEG, s) for s in slices]
        cm.append(tree_max(slices))

    cur, n1, n2, n3 = cm
    of = jnp.zeros((), jnp.bool_)
    for _ in range(TOPK - 1):
        m = jnp.max(cur, axis=1, keepdims=True)
        sel = cur == m
        of = of | jnp.any(sel & (n1 == NEG))
        cur = jnp.where(sel, n1, cur)
        n1 = jnp.where(sel, n2, n1)
        n2 = jnp.where(sel, n3, n2)
        n3 = jnp.where(sel, NEG, n3)
    thresh_fast = jnp.max(cur, axis=1, keepdims=True)
    return jax.lax.cond(of, lambda: _row_kth_full(zpre), lambda: thresh_fast)


def _body(x_ref, ew_ref, eb_ref, dw_ref, db_ref, z_ref, xh_ref):
    xb = x_ref[...]
    ew = ew_ref[...]
    zpre = jax.lax.dot_general(
        xb, ew, (((1,), (1,)), ((), ())),
        preferred_element_type=jnp.float32)
    zpre = zpre + eb_ref[...]

    thresh = _row_kth_hier(zpre)

    z = jnp.where(zpre >= thresh, jnp.maximum(zpre, 0.0), 0.0)
    z_ref[...] = z

    dw = dw_ref[...]
    xh = jax.lax.dot_general(
        z.astype(jnp.bfloat16), dw, (((1,), (1,)), ((), ())),
        preferred_element_type=jnp.float32)
    xh_ref[...] = xh + db_ref[...]


def kernel(x, enc_w, enc_b, dec_w, dec_b):
    B, N, _ = x.shape
    M = B * N
    xf = x.reshape(M, C).astype(jnp.bfloat16)
    ew = enc_w.astype(jnp.bfloat16)
    dw = dec_w.astype(jnp.bfloat16)
    eb = enc_b.reshape(1, K)
    db = dec_b.reshape(1, C)

    z, xh = pl.pallas_call(
        _body,
        grid=(M // T,),
        in_specs=[
            pl.BlockSpec((T, C), lambda i: (i, 0)),
            pl.BlockSpec((K, C), lambda i: (0, 0)),
            pl.BlockSpec((1, K), lambda i: (0, 0)),
            pl.BlockSpec((C, K), lambda i: (0, 0)),
            pl.BlockSpec((1, C), lambda i: (0, 0)),
        ],
        out_specs=[
            pl.BlockSpec((T, K), lambda i: (i, 0)),
            pl.BlockSpec((T, C), lambda i: (i, 0)),
        ],
        out_shape=[
            jax.ShapeDtypeStruct((M, K), jnp.float32),
            jax.ShapeDtypeStruct((M, C), jnp.float32),
        ],
        compiler_params=pltpu.CompilerParams(
            dimension_semantics=("parallel",)),
    )(xf, ew, eb, dw, db)
    return z.reshape(B, N, K), xh.reshape(B, N, C)


# speedup vs baseline: 27.4614x; 1.0425x over previous
import jax
import jax.numpy as jnp
from jax.experimental import pallas as pl
from jax.experimental.pallas import tpu as pltpu

C = 1024
K = 4096
TOPK = 20
T = 256
NSLICE = 16
W = K // NSLICE
NEG = float("-inf")


def _row_kth_full(zpre):
    w = zpre
    for _ in range(TOPK - 1):
        m = jnp.max(w, axis=1, keepdims=True)
        w = jnp.where(w == m, NEG, w)
    return jnp.max(w, axis=1, keepdims=True)


def _row_kth_hier(zpre):
    slices = [zpre[:, i * W:(i + 1) * W] for i in range(NSLICE)]

    def tree_max(ss):
        m = ss[0]
        for s in ss[1:]:
            m = jnp.maximum(m, s)
        return m

    cm = [tree_max(slices)]
    for _ in range(3):
        slices = [jnp.where(s == cm[-1], NEG, s) for s in slices]
        cm.append(tree_max(slices))

    cur, n1, n2, n3 = cm
    for _ in range(TOPK - 1):
        m = jnp.max(cur, axis=1, keepdims=True)
        sel = cur == m
        cur = jnp.where(sel, n1, cur)
        n1 = jnp.where(sel, n2, n1)
        n2 = jnp.where(sel, n3, n2)
        n3 = jnp.where(sel, NEG, n3)
    of = jnp.any(cur == NEG)
    thresh_fast = jnp.max(cur, axis=1, keepdims=True)
    return jax.lax.cond(of, lambda: _row_kth_full(zpre), lambda: thresh_fast)


def _body(x_ref, ew_ref, eb_ref, dw_ref, db_ref, z_ref, xh_ref):
    xb = x_ref[...]
    ew = ew_ref[...]
    zpre = jax.lax.dot_general(
        xb, ew, (((1,), (1,)), ((), ())),
        preferred_element_type=jnp.float32)
    zpre = zpre + eb_ref[...]

    thresh = _row_kth_hier(zpre)

    t2 = jnp.maximum(thresh, 0.0)
    z = jnp.where(zpre >= t2, zpre, 0.0)
    z_ref[...] = z

    dw = dw_ref[...]
    xh = jax.lax.dot_general(
        z.astype(jnp.bfloat16), dw, (((1,), (1,)), ((), ())),
        preferred_element_type=jnp.float32)
    xh_ref[...] = xh + db_ref[...]


def kernel(x, enc_w, enc_b, dec_w, dec_b):
    B, N, _ = x.shape
    M = B * N
    xf = x.reshape(M, C).astype(jnp.bfloat16)
    ew = enc_w.astype(jnp.bfloat16)
    dw = dec_w.astype(jnp.bfloat16)
    eb = enc_b.reshape(1, K)
    db = dec_b.reshape(1, C)

    z, xh = pl.pallas_call(
        _body,
        grid=(M // T,),
        in_specs=[
            pl.BlockSpec((T, C), lambda i: (i, 0)),
            pl.BlockSpec((K, C), lambda i: (0, 0)),
            pl.BlockSpec((1, K), lambda i: (0, 0)),
            pl.BlockSpec((C, K), lambda i: (0, 0)),
            pl.BlockSpec((1, C), lambda i: (0, 0)),
        ],
        out_specs=[
            pl.BlockSpec((T, K), lambda i: (i, 0)),
            pl.BlockSpec((T, C), lambda i: (i, 0)),
        ],
        out_shape=[
            jax.ShapeDtypeStruct((M, K), jnp.float32),
            jax.ShapeDtypeStruct((M, C), jnp.float32),
        ],
        compiler_params=pltpu.CompilerParams(
            dimension_semantics=("parallel",)),
    )(xf, ew, eb, dw, db)
    return z.reshape(B, N, K), xh.reshape(B, N, C)
